# baseline (speedup 1.0000x reference)
import functools
import math

import jax
import jax.numpy as jnp
from jax import lax
from jax.experimental import pallas as pl
from jax.experimental.pallas import tpu as pltpu
from jax.experimental.pallas import tpu_sc as plsc

N_NODES = 50000
N_EDGES = 800000
N_FEATURES = 128
N_BASIS = 16
R_CUT = 5.0

NW = 32

ROWS_PER_W = 1568
ROW_CHUNK = 224
N_ROW_CHUNKS = ROWS_PER_W // ROW_CHUNK

EDGES_PER_TILE = 25000
EDGES_PER_W = 25088
EDGE_CHUNK = 1792
N_EDGE_CHUNKS = EDGES_PER_W // EDGE_CHUNK
VECS_PER_CHUNK = EDGE_CHUNK // 16


def _sc_atom_body(z_hbm, posf_hbm, emb_hbm, atom_hbm, px_hbm, py_hbm, pz_hbm,
                  idx_v, rows_v, stage_v, cx_v, cy_v, cz_v, sem):
    cid = lax.axis_index("c")
    sid = lax.axis_index("s")
    wid = sid * 2 + cid
    row0 = jnp.minimum(wid * ROWS_PER_W, N_NODES - ROWS_PER_W)

    pltpu.sync_copy(posf_hbm.at[pl.ds(row0 * 3, 3 * ROWS_PER_W)], stage_v)
    lane3 = lax.iota(jnp.int32, 16) * 3

    def dei(i, carry):
        base = i * 48
        cx_v[pl.ds(i * 16, 16)] = plsc.load_gather(stage_v, [lane3 + base])
        cy_v[pl.ds(i * 16, 16)] = plsc.load_gather(stage_v,
                                                   [lane3 + (base + 1)])
        cz_v[pl.ds(i * 16, 16)] = plsc.load_gather(stage_v,
                                                   [lane3 + (base + 2)])
        return carry

    lax.fori_loop(0, ROWS_PER_W // 16, dei, 0)
    pltpu.sync_copy(cx_v, px_hbm.at[pl.ds(row0, ROWS_PER_W)])
    pltpu.sync_copy(cy_v, py_hbm.at[pl.ds(row0, ROWS_PER_W)])
    pltpu.sync_copy(cz_v, pz_hbm.at[pl.ds(row0, ROWS_PER_W)])

    for k in range(N_ROW_CHUNKS):
        base = row0 + k * ROW_CHUNK
        pltpu.sync_copy(z_hbm.at[pl.ds(base, ROW_CHUNK)], idx_v)
        pltpu.async_copy(emb_hbm.at[idx_v], rows_v, sem).wait()
        pltpu.sync_copy(rows_v, atom_hbm.at[pl.ds(base, ROW_CHUNK)])


_sc_atom = functools.partial(
    pl.kernel,
    out_type=(
        jax.ShapeDtypeStruct((N_NODES, N_FEATURES), jnp.float32),
        jax.ShapeDtypeStruct((N_NODES,), jnp.float32),
        jax.ShapeDtypeStruct((N_NODES,), jnp.float32),
        jax.ShapeDtypeStruct((N_NODES,), jnp.float32),
    ),
    mesh=plsc.VectorSubcoreMesh(core_axis_name="c", subcore_axis_name="s"),
    compiler_params=pltpu.CompilerParams(needs_layout_passes=False),
    scratch_types=[
        pltpu.VMEM((ROW_CHUNK,), jnp.int32),
        pltpu.VMEM((ROW_CHUNK, N_FEATURES), jnp.float32),
        pltpu.VMEM((3 * ROWS_PER_W,), jnp.float32),
        pltpu.VMEM((ROWS_PER_W,), jnp.float32),
        pltpu.VMEM((ROWS_PER_W,), jnp.float32),
        pltpu.VMEM((ROWS_PER_W,), jnp.float32),
        pltpu.SemaphoreType.DMA,
    ],
)(_sc_atom_body)


def _sc_edge_body(px_hbm, py_hbm, pz_hbm, edgef_hbm,
                  disp_hbm, dx_hbm, dy_hbm, dz_hbm,
                  pcomp_v, src_v, dst_v, out_v, disp_v):
    cid = lax.axis_index("c")
    sid = lax.axis_index("s")
    wid = sid * 2 + cid
    eb = jnp.minimum(wid * EDGES_PER_TILE, N_EDGES - EDGES_PER_W)
    lane3 = lax.iota(jnp.int32, 16) * 3
    for ci, (p_hbm, o_hbm) in enumerate(
            ((px_hbm, dx_hbm), (py_hbm, dy_hbm), (pz_hbm, dz_hbm))):
        pltpu.sync_copy(p_hbm, pcomp_v)
        for k in range(N_EDGE_CHUNKS):
            cb = k * EDGE_CHUNK
            pltpu.sync_copy(edgef_hbm.at[pl.ds(eb + cb, EDGE_CHUNK)], src_v)
            pltpu.sync_copy(
                edgef_hbm.at[pl.ds(N_EDGES + eb + cb, EDGE_CHUNK)], dst_v)

            def body(i, carry, _cb=cb, _ci=ci):
                s = src_v[pl.ds(i * 16, 16)]
                t = dst_v[pl.ds(i * 16, 16)]
                d = plsc.load_gather(pcomp_v, [s]) - plsc.load_gather(
                    pcomp_v, [t])
                out_v[pl.ds(i * 16, 16)] = d
                plsc.store_scatter(
                    disp_v, [lane3 + ((_cb + i * 16) * 3 + _ci)], d)
                return carry

            lax.fori_loop(0, VECS_PER_CHUNK, body, 0)
            pltpu.sync_copy(out_v, o_hbm.at[pl.ds(eb + cb, EDGE_CHUNK)])
    pltpu.sync_copy(disp_v, disp_hbm.at[pl.ds(eb * 3, 3 * EDGES_PER_W)])


_sc_edge = functools.partial(
    pl.kernel,
    out_type=(
        jax.ShapeDtypeStruct((3 * N_EDGES,), jnp.float32),
        jax.ShapeDtypeStruct((N_EDGES,), jnp.float32),
        jax.ShapeDtypeStruct((N_EDGES,), jnp.float32),
        jax.ShapeDtypeStruct((N_EDGES,), jnp.float32),
    ),
    mesh=plsc.VectorSubcoreMesh(core_axis_name="c", subcore_axis_name="s"),
    compiler_params=pltpu.CompilerParams(needs_layout_passes=False),
    scratch_types=[
        pltpu.VMEM((N_NODES,), jnp.float32),
        pltpu.VMEM((EDGE_CHUNK,), jnp.int32),
        pltpu.VMEM((EDGE_CHUNK,), jnp.int32),
        pltpu.VMEM((EDGE_CHUNK,), jnp.float32),
        pltpu.VMEM((3 * EDGES_PER_W,), jnp.float32),
    ],
)(_sc_edge_body)


_EDGE_BLK = 6400
_ROWS = _EDGE_BLK // 8
_N_EDGE_BLKS = N_EDGES // _EDGE_BLK
_CENTER_STEP = R_CUT / (N_BASIS - 1)
_GAMMA = 1.0 / (_CENTER_STEP * _CENTER_STEP)


_COS_POLY = (0.9999999890590233, -4.934801124863485, 4.058694841243486,
             -1.3351584301699686, 0.23502980840174797,
             -0.025358983640522026, 0.001593910683660976)


def _tc_dense_body(dx_ref, dy_ref, dz_ref, d_ref, cut_ref):
    x = dx_ref[...]
    y = dy_ref[...]
    z = dz_ref[...]
    d2 = x * x + y * y + z * z
    dist = jnp.sqrt(d2)
    v = d2 * (1.0 / (R_CUT * R_CUT))
    p = jnp.float32(_COS_POLY[6])
    for coef in _COS_POLY[5::-1]:
        p = p * v + jnp.float32(coef)
    cut = 0.5 * (p + 1.0)
    d_ref[...] = dist
    cut_ref[...] = cut * (v < 1.0).astype(jnp.float32)


def _tc_rep_body(d8_ref, cut8_ref, out_ref):
    idx = lax.broadcasted_iota(jnp.int32, (_ROWS, 128), 1) // N_BASIS
    dist = jnp.take_along_axis(d8_ref[...], idx, axis=1)
    cut = jnp.take_along_axis(cut8_ref[...], idx, axis=1)
    centers = (lax.broadcasted_iota(jnp.int32, (_ROWS, 128), 1) % N_BASIS
               ).astype(jnp.float32) * _CENTER_STEP
    delta = dist - centers
    out_ref[...] = cut * jnp.exp(-_GAMMA * delta * delta)


_DENSE_ROWS = N_EDGES // 128
_DENSE_BLK = 800
_N_DENSE_BLKS = -(-_DENSE_ROWS // _DENSE_BLK)


def _dist_edge(dx, dy, dz):
    dspec = pl.BlockSpec((_DENSE_BLK, 128), lambda i: (i, 0))
    dist, cut = pl.pallas_call(
        _tc_dense_body,
        grid=(_N_DENSE_BLKS,),
        in_specs=[dspec, dspec, dspec],
        out_specs=[dspec, dspec],
        out_shape=[jax.ShapeDtypeStruct((_DENSE_ROWS, 128), jnp.float32),
                   jax.ShapeDtypeStruct((_DENSE_ROWS, 128), jnp.float32)],
    )(dx.reshape(_DENSE_ROWS, 128), dy.reshape(_DENSE_ROWS, 128),
      dz.reshape(_DENSE_ROWS, 128))
    spec = pl.BlockSpec((_ROWS, 8), lambda i: (i, 0))
    flat = pl.pallas_call(
        _tc_rep_body,
        grid=(_N_EDGE_BLKS,),
        in_specs=[spec, spec],
        out_specs=pl.BlockSpec((_ROWS, 128), lambda i: (i, 0)),
        out_shape=jax.ShapeDtypeStruct((N_EDGES // 8, 128), jnp.float32),
    )(dist.reshape(N_EDGES // 8, 8), cut.reshape(N_EDGES // 8, 8))
    return flat.reshape(N_EDGES, N_BASIS)


def kernel(z, pos, edge_index, emb_table):
    pos_flat = pos.reshape(3 * N_NODES)
    edge_flat = edge_index.reshape(2 * N_EDGES)
    atom_node, px, py, pz = _sc_atom(z, pos_flat, emb_table)
    disp_flat, dx, dy, dz = _sc_edge(px, py, pz, edge_flat)
    disp_edge = disp_flat.reshape(N_EDGES, 3)
    dist_edge = _dist_edge(dx, dy, dz)
    zeros = jnp.zeros((N_NODES, 3, N_FEATURES), dtype=jnp.float32)
    return (atom_node, zeros, zeros, disp_edge, dist_edge)


# speedup vs baseline: 1.0343x; 1.0343x over previous
import functools
import math

import jax
import jax.numpy as jnp
from jax import lax
from jax.experimental import pallas as pl
from jax.experimental.pallas import tpu as pltpu
from jax.experimental.pallas import tpu_sc as plsc

N_NODES = 50000
N_EDGES = 800000
N_FEATURES = 128
N_BASIS = 16
R_CUT = 5.0

NW = 32

ROWS_PER_W = 1568
ROW_CHUNK = 224
N_ROW_CHUNKS = ROWS_PER_W // ROW_CHUNK

EDGES_PER_W = 25088
EDGE_CHUNK = 1792
N_EDGE_CHUNKS = EDGES_PER_W // EDGE_CHUNK
VECS_PER_CHUNK = EDGE_CHUNK // 16


def _sc_atom_body(z_hbm, emb_hbm, atom_hbm, idx_v, rows_v, sem):
    cid = lax.axis_index("c")
    sid = lax.axis_index("s")
    wid = sid * 2 + cid
    row0 = jnp.minimum(wid * ROWS_PER_W, N_NODES - ROWS_PER_W)
    for k in range(N_ROW_CHUNKS):
        base = row0 + k * ROW_CHUNK
        pltpu.sync_copy(z_hbm.at[pl.ds(base, ROW_CHUNK)], idx_v)
        pltpu.async_copy(emb_hbm.at[idx_v], rows_v, sem).wait()
        pltpu.sync_copy(rows_v, atom_hbm.at[pl.ds(base, ROW_CHUNK)])


_sc_atom = functools.partial(
    pl.kernel,
    out_type=jax.ShapeDtypeStruct((N_NODES, N_FEATURES), jnp.float32),
    mesh=plsc.VectorSubcoreMesh(core_axis_name="c", subcore_axis_name="s"),
    compiler_params=pltpu.CompilerParams(needs_layout_passes=False),
    scratch_types=[
        pltpu.VMEM((ROW_CHUNK,), jnp.int32),
        pltpu.VMEM((ROW_CHUNK, N_FEATURES), jnp.float32),
        pltpu.SemaphoreType.DMA,
    ],
)(_sc_atom_body)


def _prep_pos_body(pos_ref, px_ref, py_ref, pz_ref):
    p = pos_ref[...]
    px_ref[...] = p[:, 0]
    py_ref[...] = p[:, 1]
    pz_ref[...] = p[:, 2]


def _prep_edge_body(e_ref, src_ref, dst_ref):
    e = e_ref[...]
    src_ref[...] = e[0, :]
    dst_ref[...] = e[1, :]


_POS_BLK = 5120


def _prep_pos(pos):
    nblk = -(-N_NODES // _POS_BLK)
    o = jax.ShapeDtypeStruct((N_NODES,), jnp.float32)
    ospec = pl.BlockSpec((_POS_BLK,), lambda i: (i,))
    return pl.pallas_call(
        _prep_pos_body,
        grid=(nblk,),
        in_specs=[pl.BlockSpec((_POS_BLK, 3), lambda i: (i, 0))],
        out_specs=[ospec, ospec, ospec],
        out_shape=[o, o, o],
    )(pos)


_EIDX_BLK = 102400


def _prep_edge(edge_index):
    nblk = -(-N_EDGES // _EIDX_BLK)
    o = jax.ShapeDtypeStruct((N_EDGES,), jnp.int32)
    ospec = pl.BlockSpec((_EIDX_BLK,), lambda i: (i,))
    return pl.pallas_call(
        _prep_edge_body,
        grid=(nblk,),
        in_specs=[pl.BlockSpec((2, _EIDX_BLK), lambda i: (0, i))],
        out_specs=[ospec, ospec],
        out_shape=[o, o],
    )(edge_index)


def _sc_edge_body(px_hbm, py_hbm, pz_hbm, src_hbm, dst_hbm,
                  disp_hbm, dx_hbm, dy_hbm, dz_hbm,
                  pcomp_v, src_v, dst_v, out_v, disp_v):
    cid = lax.axis_index("c")
    sid = lax.axis_index("s")
    wid = sid * 2 + cid
    eb = jnp.minimum(wid * EDGES_PER_W, N_EDGES - EDGES_PER_W)
    lane3 = lax.iota(jnp.int32, 16) * 3
    for ci, (p_hbm, o_hbm) in enumerate(
            ((px_hbm, dx_hbm), (py_hbm, dy_hbm), (pz_hbm, dz_hbm))):
        pltpu.sync_copy(p_hbm, pcomp_v)
        for k in range(N_EDGE_CHUNKS):
            cb = k * EDGE_CHUNK
            pltpu.sync_copy(src_hbm.at[pl.ds(eb + cb, EDGE_CHUNK)], src_v)
            pltpu.sync_copy(dst_hbm.at[pl.ds(eb + cb, EDGE_CHUNK)], dst_v)

            def body(i, carry, _cb=cb, _ci=ci):
                s = src_v[pl.ds(i * 16, 16)]
                t = dst_v[pl.ds(i * 16, 16)]
                d = plsc.load_gather(pcomp_v, [s]) - plsc.load_gather(
                    pcomp_v, [t])
                out_v[pl.ds(i * 16, 16)] = d
                plsc.store_scatter(
                    disp_v, [lane3 + ((_cb + i * 16) * 3 + _ci)], d)
                return carry

            lax.fori_loop(0, VECS_PER_CHUNK, body, 0)
            pltpu.sync_copy(out_v, o_hbm.at[pl.ds(eb + cb, EDGE_CHUNK)])
    pltpu.sync_copy(disp_v, disp_hbm.at[pl.ds(eb * 3, 3 * EDGES_PER_W)])


_sc_edge = functools.partial(
    pl.kernel,
    out_type=(
        jax.ShapeDtypeStruct((3 * N_EDGES,), jnp.float32),
        jax.ShapeDtypeStruct((N_EDGES,), jnp.float32),
        jax.ShapeDtypeStruct((N_EDGES,), jnp.float32),
        jax.ShapeDtypeStruct((N_EDGES,), jnp.float32),
    ),
    mesh=plsc.VectorSubcoreMesh(core_axis_name="c", subcore_axis_name="s"),
    compiler_params=pltpu.CompilerParams(needs_layout_passes=False),
    scratch_types=[
        pltpu.VMEM((N_NODES,), jnp.float32),
        pltpu.VMEM((EDGE_CHUNK,), jnp.int32),
        pltpu.VMEM((EDGE_CHUNK,), jnp.int32),
        pltpu.VMEM((EDGE_CHUNK,), jnp.float32),
        pltpu.VMEM((3 * EDGES_PER_W,), jnp.float32),
    ],
)(_sc_edge_body)


_EDGE_BLK = 6400
_ROWS = _EDGE_BLK // 8
_N_EDGE_BLKS = N_EDGES // _EDGE_BLK
_CENTER_STEP = R_CUT / (N_BASIS - 1)
_GAMMA = 1.0 / (_CENTER_STEP * _CENTER_STEP)


_COS_POLY = (0.9999999890590233, -4.934801124863485, 4.058694841243486,
             -1.3351584301699686, 0.23502980840174797,
             -0.025358983640522026, 0.001593910683660976)


def _tc_dense_body(dx_ref, dy_ref, dz_ref, d_ref, cut_ref):
    x = dx_ref[...]
    y = dy_ref[...]
    z = dz_ref[...]
    d2 = x * x + y * y + z * z
    dist = jnp.sqrt(d2)
    v = d2 * (1.0 / (R_CUT * R_CUT))
    p = jnp.float32(_COS_POLY[6])
    for coef in _COS_POLY[5::-1]:
        p = p * v + jnp.float32(coef)
    cut = 0.5 * (p + 1.0)
    d_ref[...] = dist
    cut_ref[...] = cut * (v < 1.0).astype(jnp.float32)


def _tc_rep_body(d8_ref, cut8_ref, out_ref):
    idx = lax.broadcasted_iota(jnp.int32, (_ROWS, 128), 1) // N_BASIS
    dist = jnp.take_along_axis(d8_ref[...], idx, axis=1)
    cut = jnp.take_along_axis(cut8_ref[...], idx, axis=1)
    centers = (lax.broadcasted_iota(jnp.int32, (_ROWS, 128), 1) % N_BASIS
               ).astype(jnp.float32) * _CENTER_STEP
    delta = dist - centers
    out_ref[...] = cut * jnp.exp(-_GAMMA * delta * delta)


_DENSE_ROWS = N_EDGES // 128
_DENSE_BLK = 800
_N_DENSE_BLKS = -(-_DENSE_ROWS // _DENSE_BLK)


def _dist_edge(dx, dy, dz):
    dspec = pl.BlockSpec((_DENSE_BLK, 128), lambda i: (i, 0))
    dist, cut = pl.pallas_call(
        _tc_dense_body,
        grid=(_N_DENSE_BLKS,),
        in_specs=[dspec, dspec, dspec],
        out_specs=[dspec, dspec],
        out_shape=[jax.ShapeDtypeStruct((_DENSE_ROWS, 128), jnp.float32),
                   jax.ShapeDtypeStruct((_DENSE_ROWS, 128), jnp.float32)],
    )(dx.reshape(_DENSE_ROWS, 128), dy.reshape(_DENSE_ROWS, 128),
      dz.reshape(_DENSE_ROWS, 128))
    spec = pl.BlockSpec((_ROWS, 8), lambda i: (i, 0))
    flat = pl.pallas_call(
        _tc_rep_body,
        grid=(_N_EDGE_BLKS,),
        in_specs=[spec, spec],
        out_specs=pl.BlockSpec((_ROWS, 128), lambda i: (i, 0)),
        out_shape=jax.ShapeDtypeStruct((N_EDGES // 8, 128), jnp.float32),
    )(dist.reshape(N_EDGES // 8, 8), cut.reshape(N_EDGES // 8, 8))
    return flat.reshape(N_EDGES, N_BASIS)


def kernel(z, pos, edge_index, emb_table):
    px, py, pz = _prep_pos(pos)
    src, dst = _prep_edge(edge_index)
    atom_node = _sc_atom(z, emb_table)
    disp_flat, dx, dy, dz = _sc_edge(px, py, pz, src, dst)
    disp_edge = disp_flat.reshape(N_EDGES, 3)
    dist_edge = _dist_edge(dx, dy, dz)
    zeros = jnp.zeros((N_NODES, 3, N_FEATURES), dtype=jnp.float32)
    return (atom_node, zeros, zeros, disp_edge, dist_edge)


# speedup vs baseline: 3.6323x; 3.5117x over previous
import functools
import math

import jax
import jax.numpy as jnp
from jax import lax
from jax.experimental import pallas as pl
from jax.experimental.pallas import tpu as pltpu
from jax.experimental.pallas import tpu_sc as plsc

N_NODES = 50000
N_EDGES = 800000
N_FEATURES = 128
N_BASIS = 16
R_CUT = 5.0

NW = 32

ROWS_PER_W = 1568
ROW_CHUNK = 224
N_ROW_CHUNKS = ROWS_PER_W // ROW_CHUNK

EDGES_PER_W = 25088
EDGE_CHUNK = 1792
N_EDGE_CHUNKS = EDGES_PER_W // EDGE_CHUNK
VECS_PER_CHUNK = EDGE_CHUNK // 16


def _sc_atom_body(z_hbm, emb_hbm, atom_hbm, idx_v, rows_v, sem):
    cid = lax.axis_index("c")
    sid = lax.axis_index("s")
    wid = sid * 2 + cid
    row0 = jnp.minimum(wid * ROWS_PER_W, N_NODES - ROWS_PER_W)
    for k in range(N_ROW_CHUNKS):
        base = row0 + k * ROW_CHUNK
        pltpu.sync_copy(z_hbm.at[pl.ds(base, ROW_CHUNK)], idx_v)
        pltpu.async_copy(emb_hbm.at[idx_v], rows_v, sem).wait()
        pltpu.sync_copy(rows_v, atom_hbm.at[pl.ds(base, ROW_CHUNK)])


_sc_atom = functools.partial(
    pl.kernel,
    out_type=jax.ShapeDtypeStruct((N_NODES, N_FEATURES), jnp.float32),
    mesh=plsc.VectorSubcoreMesh(core_axis_name="c", subcore_axis_name="s"),
    compiler_params=pltpu.CompilerParams(needs_layout_passes=False),
    scratch_types=[
        pltpu.VMEM((ROW_CHUNK,), jnp.int32),
        pltpu.VMEM((ROW_CHUNK, N_FEATURES), jnp.float32),
        pltpu.SemaphoreType.DMA,
    ],
)(_sc_atom_body)


def _prep_pos_body(pos_ref, px_ref, py_ref, pz_ref):
    p = pos_ref[...]
    px_ref[...] = p[:, 0]
    py_ref[...] = p[:, 1]
    pz_ref[...] = p[:, 2]


def _prep_edge_body(e_ref, src_ref, dst_ref):
    e = e_ref[...]
    src_ref[...] = e[0, :]
    dst_ref[...] = e[1, :]


_POS_BLK = 5120


def _prep_pos(pos):
    nblk = -(-N_NODES // _POS_BLK)
    o = jax.ShapeDtypeStruct((N_NODES,), jnp.float32)
    ospec = pl.BlockSpec((_POS_BLK,), lambda i: (i,))
    return pl.pallas_call(
        _prep_pos_body,
        grid=(nblk,),
        in_specs=[pl.BlockSpec((_POS_BLK, 3), lambda i: (i, 0))],
        out_specs=[ospec, ospec, ospec],
        out_shape=[o, o, o],
    )(pos)


_EIDX_BLK = 102400


def _prep_edge(edge_index):
    nblk = -(-N_EDGES // _EIDX_BLK)
    o = jax.ShapeDtypeStruct((N_EDGES,), jnp.int32)
    ospec = pl.BlockSpec((_EIDX_BLK,), lambda i: (i,))
    return pl.pallas_call(
        _prep_edge_body,
        grid=(nblk,),
        in_specs=[pl.BlockSpec((2, _EIDX_BLK), lambda i: (0, i))],
        out_specs=[ospec, ospec],
        out_shape=[o, o],
    )(edge_index)


def _sc_edge_body(px_hbm, py_hbm, pz_hbm, src_hbm, dst_hbm,
                  dx_hbm, dy_hbm, dz_hbm,
                  pcomp_v, src_v, dst_v, out_v):
    cid = lax.axis_index("c")
    sid = lax.axis_index("s")
    wid = sid * 2 + cid
    eb = jnp.minimum(wid * EDGES_PER_W, N_EDGES - EDGES_PER_W)
    pltpu.sync_copy(src_hbm.at[pl.ds(eb, EDGES_PER_W)], src_v)
    pltpu.sync_copy(dst_hbm.at[pl.ds(eb, EDGES_PER_W)], dst_v)
    for p_hbm, o_hbm in ((px_hbm, dx_hbm), (py_hbm, dy_hbm), (pz_hbm, dz_hbm)):
        pltpu.sync_copy(p_hbm, pcomp_v)

        def body(i, carry):
            s = src_v[pl.ds(i * 16, 16)]
            t = dst_v[pl.ds(i * 16, 16)]
            d = plsc.load_gather(pcomp_v, [s]) - plsc.load_gather(pcomp_v, [t])
            out_v[pl.ds(i * 16, 16)] = d
            return carry

        lax.fori_loop(0, EDGES_PER_W // 16, body, 0)
        pltpu.sync_copy(out_v, o_hbm.at[pl.ds(eb, EDGES_PER_W)])


_sc_edge = functools.partial(
    pl.kernel,
    out_type=(
        jax.ShapeDtypeStruct((N_EDGES,), jnp.float32),
        jax.ShapeDtypeStruct((N_EDGES,), jnp.float32),
        jax.ShapeDtypeStruct((N_EDGES,), jnp.float32),
    ),
    mesh=plsc.VectorSubcoreMesh(core_axis_name="c", subcore_axis_name="s"),
    compiler_params=pltpu.CompilerParams(needs_layout_passes=False),
    scratch_types=[
        pltpu.VMEM((N_NODES,), jnp.float32),
        pltpu.VMEM((EDGES_PER_W,), jnp.int32),
        pltpu.VMEM((EDGES_PER_W,), jnp.int32),
        pltpu.VMEM((EDGES_PER_W,), jnp.float32),
    ],
)(_sc_edge_body)


_EDGE_BLK = 5120
_N_EDGE_BLKS = -(-N_EDGES // _EDGE_BLK)
_CENTER_STEP = R_CUT / (N_BASIS - 1)
_GAMMA = 1.0 / (_CENTER_STEP * _CENTER_STEP)


_COS_POLY = (0.9999999890590233, -4.934801124863485, 4.058694841243486,
             -1.3351584301699686, 0.23502980840174797,
             -0.025358983640522026, 0.001593910683660976)


def _tc_dist_body(dx_ref, dy_ref, dz_ref, out_ref):
    x = dx_ref[...]
    y = dy_ref[...]
    z = dz_ref[...]
    d2 = x * x + y * y + z * z
    dist = jnp.sqrt(d2)
    v = d2 * (1.0 / (R_CUT * R_CUT))
    p = jnp.float32(_COS_POLY[6])
    for coef in _COS_POLY[5::-1]:
        p = p * v + jnp.float32(coef)
    cut = 0.5 * (p + 1.0) * (v < 1.0).astype(jnp.float32)
    cut2 = jnp.broadcast_to(cut.reshape(1, _EDGE_BLK), (N_BASIS, _EDGE_BLK))
    dist2 = jnp.broadcast_to(dist.reshape(1, _EDGE_BLK), (N_BASIS, _EDGE_BLK))
    centers = lax.broadcasted_iota(jnp.int32, (N_BASIS, _EDGE_BLK), 0).astype(
        jnp.float32) * _CENTER_STEP
    delta = dist2 - centers
    out_ref[...] = cut2 * jnp.exp(-_GAMMA * delta * delta)


def _dist_edge_t(dx, dy, dz):
    spec = pl.BlockSpec((_EDGE_BLK,), lambda i: (i,))
    return pl.pallas_call(
        _tc_dist_body,
        grid=(_N_EDGE_BLKS,),
        in_specs=[spec, spec, spec],
        out_specs=pl.BlockSpec((N_BASIS, _EDGE_BLK), lambda i: (0, i)),
        out_shape=jax.ShapeDtypeStruct((N_BASIS, N_EDGES), jnp.float32),
    )(dx, dy, dz)


def kernel(z, pos, edge_index, emb_table):
    px, py, pz = _prep_pos(pos)
    src, dst = _prep_edge(edge_index)
    atom_node = _sc_atom(z, emb_table)
    dx, dy, dz = _sc_edge(px, py, pz, src, dst)
    disp_edge = jnp.stack([dx, dy, dz], axis=1)
    dist_edge = _dist_edge_t(dx, dy, dz).T
    zeros = jnp.zeros((N_NODES, 3, N_FEATURES), dtype=jnp.float32)
    return (atom_node, zeros, zeros, disp_edge, dist_edge)


# speedup vs baseline: 3.8195x; 1.0515x over previous
import functools
import math

import jax
import jax.numpy as jnp
from jax import lax
from jax.experimental import pallas as pl
from jax.experimental.pallas import tpu as pltpu
from jax.experimental.pallas import tpu_sc as plsc

N_NODES = 50000
N_EDGES = 800000
N_FEATURES = 128
N_BASIS = 16
R_CUT = 5.0

NW = 32

ROWS_PER_W = 1568
ROW_CHUNK = 224
N_ROW_CHUNKS = ROWS_PER_W // ROW_CHUNK

EDGES_PER_W = 25088
EDGE_CHUNK = 1792
N_EDGE_CHUNKS = EDGES_PER_W // EDGE_CHUNK
VECS_PER_CHUNK = EDGE_CHUNK // 16


def _sc_atom_body(z_hbm, emb_hbm, atom_hbm, idx_v, rows_v0, rows_v1,
                  gsem0, gsem1, osem0, osem1):
    cid = lax.axis_index("c")
    sid = lax.axis_index("s")
    wid = sid * 2 + cid
    row0 = jnp.minimum(wid * ROWS_PER_W, N_NODES - ROWS_PER_W)
    pltpu.sync_copy(z_hbm.at[pl.ds(row0, ROWS_PER_W)], idx_v)

    rows = (rows_v0, rows_v1)
    gsem = (gsem0, gsem1)
    osem = (osem0, osem1)
    gcp = [None, None]
    ocp = [None, None]
    gcp[0] = pltpu.async_copy(
        emb_hbm.at[idx_v.at[pl.ds(0, ROW_CHUNK)]], rows[0], gsem[0])
    for k in range(N_ROW_CHUNKS):
        b = k % 2
        nb = (k + 1) % 2
        if k + 1 < N_ROW_CHUNKS:
            if ocp[nb] is not None:
                ocp[nb].wait()
            gcp[nb] = pltpu.async_copy(
                emb_hbm.at[idx_v.at[pl.ds((k + 1) * ROW_CHUNK, ROW_CHUNK)]],
                rows[nb], gsem[nb])
        gcp[b].wait()
        ocp[b] = pltpu.async_copy(
            rows[b], atom_hbm.at[pl.ds(row0 + k * ROW_CHUNK, ROW_CHUNK)],
            osem[b])
    ocp[0].wait()
    ocp[1].wait()


_sc_atom = functools.partial(
    pl.kernel,
    out_type=jax.ShapeDtypeStruct((N_NODES, N_FEATURES), jnp.float32),
    mesh=plsc.VectorSubcoreMesh(core_axis_name="c", subcore_axis_name="s"),
    compiler_params=pltpu.CompilerParams(needs_layout_passes=False),
    scratch_types=[
        pltpu.VMEM((ROWS_PER_W,), jnp.int32),
        pltpu.VMEM((ROW_CHUNK, N_FEATURES), jnp.float32),
        pltpu.VMEM((ROW_CHUNK, N_FEATURES), jnp.float32),
        pltpu.SemaphoreType.DMA,
        pltpu.SemaphoreType.DMA,
        pltpu.SemaphoreType.DMA,
        pltpu.SemaphoreType.DMA,
    ],
)(_sc_atom_body)


def _prep_pos_body(pos_ref, px_ref, py_ref, pz_ref):
    p = pos_ref[...]
    px_ref[...] = p[:, 0]
    py_ref[...] = p[:, 1]
    pz_ref[...] = p[:, 2]


def _prep_edge_body(e_ref, src_ref, dst_ref):
    e = e_ref[...]
    src_ref[...] = e[0, :]
    dst_ref[...] = e[1, :]


_POS_BLK = 5120


def _prep_pos(pos):
    nblk = -(-N_NODES // _POS_BLK)
    o = jax.ShapeDtypeStruct((N_NODES,), jnp.float32)
    ospec = pl.BlockSpec((_POS_BLK,), lambda i: (i,))
    return pl.pallas_call(
        _prep_pos_body,
        grid=(nblk,),
        in_specs=[pl.BlockSpec((_POS_BLK, 3), lambda i: (i, 0))],
        out_specs=[ospec, ospec, ospec],
        out_shape=[o, o, o],
    )(pos)


_EIDX_BLK = 102400


def _prep_edge(edge_index):
    nblk = -(-N_EDGES // _EIDX_BLK)
    o = jax.ShapeDtypeStruct((N_EDGES,), jnp.int32)
    ospec = pl.BlockSpec((_EIDX_BLK,), lambda i: (i,))
    return pl.pallas_call(
        _prep_edge_body,
        grid=(nblk,),
        in_specs=[pl.BlockSpec((2, _EIDX_BLK), lambda i: (0, i))],
        out_specs=[ospec, ospec],
        out_shape=[o, o],
    )(edge_index)


def _sc_edge_body(px_hbm, py_hbm, pz_hbm, src_hbm, dst_hbm,
                  dx_hbm, dy_hbm, dz_hbm,
                  pcomp_v, src_v, dst_v, out_v):
    cid = lax.axis_index("c")
    sid = lax.axis_index("s")
    wid = sid * 2 + cid
    eb = jnp.minimum(wid * EDGES_PER_W, N_EDGES - EDGES_PER_W)
    pltpu.sync_copy(src_hbm.at[pl.ds(eb, EDGES_PER_W)], src_v)
    pltpu.sync_copy(dst_hbm.at[pl.ds(eb, EDGES_PER_W)], dst_v)
    for p_hbm, o_hbm in ((px_hbm, dx_hbm), (py_hbm, dy_hbm), (pz_hbm, dz_hbm)):
        pltpu.sync_copy(p_hbm, pcomp_v)

        @plsc.parallel_loop(0, EDGES_PER_W, step=16, unroll=8)
        def body(i):
            s = src_v[pl.ds(i, 16)]
            t = dst_v[pl.ds(i, 16)]
            out_v[pl.ds(i, 16)] = (plsc.load_gather(pcomp_v, [s])
                                   - plsc.load_gather(pcomp_v, [t]))

        pltpu.sync_copy(out_v, o_hbm.at[pl.ds(eb, EDGES_PER_W)])


_sc_edge = functools.partial(
    pl.kernel,
    out_type=(
        jax.ShapeDtypeStruct((N_EDGES,), jnp.float32),
        jax.ShapeDtypeStruct((N_EDGES,), jnp.float32),
        jax.ShapeDtypeStruct((N_EDGES,), jnp.float32),
    ),
    mesh=plsc.VectorSubcoreMesh(core_axis_name="c", subcore_axis_name="s"),
    compiler_params=pltpu.CompilerParams(needs_layout_passes=False),
    scratch_types=[
        pltpu.VMEM((N_NODES,), jnp.float32),
        pltpu.VMEM((EDGES_PER_W,), jnp.int32),
        pltpu.VMEM((EDGES_PER_W,), jnp.int32),
        pltpu.VMEM((EDGES_PER_W,), jnp.float32),
    ],
)(_sc_edge_body)


_EDGE_BLK = 5120
_N_EDGE_BLKS = -(-N_EDGES // _EDGE_BLK)
_CENTER_STEP = R_CUT / (N_BASIS - 1)
_GAMMA = 1.0 / (_CENTER_STEP * _CENTER_STEP)


_COS_POLY = (0.9999999890590233, -4.934801124863485, 4.058694841243486,
             -1.3351584301699686, 0.23502980840174797,
             -0.025358983640522026, 0.001593910683660976)


def _tc_dist_body(dx_ref, dy_ref, dz_ref, out_ref):
    x = dx_ref[...]
    y = dy_ref[...]
    z = dz_ref[...]
    d2 = x * x + y * y + z * z
    dist = jnp.sqrt(d2)
    v = d2 * (1.0 / (R_CUT * R_CUT))
    p = jnp.float32(_COS_POLY[6])
    for coef in _COS_POLY[5::-1]:
        p = p * v + jnp.float32(coef)
    cut = 0.5 * (p + 1.0) * (v < 1.0).astype(jnp.float32)
    cut2 = jnp.broadcast_to(cut.reshape(1, _EDGE_BLK), (N_BASIS, _EDGE_BLK))
    dist2 = jnp.broadcast_to(dist.reshape(1, _EDGE_BLK), (N_BASIS, _EDGE_BLK))
    centers = lax.broadcasted_iota(jnp.int32, (N_BASIS, _EDGE_BLK), 0).astype(
        jnp.float32) * _CENTER_STEP
    delta = dist2 - centers
    out_ref[...] = cut2 * jnp.exp(-_GAMMA * delta * delta)


def _dist_edge_t(dx, dy, dz):
    spec = pl.BlockSpec((_EDGE_BLK,), lambda i: (i,))
    return pl.pallas_call(
        _tc_dist_body,
        grid=(_N_EDGE_BLKS,),
        in_specs=[spec, spec, spec],
        out_specs=pl.BlockSpec((N_BASIS, _EDGE_BLK), lambda i: (0, i)),
        out_shape=jax.ShapeDtypeStruct((N_BASIS, N_EDGES), jnp.float32),
    )(dx, dy, dz)


def kernel(z, pos, edge_index, emb_table):
    px, py, pz = _prep_pos(pos)
    src, dst = _prep_edge(edge_index)
    atom_node = _sc_atom(z, emb_table)
    dx, dy, dz = _sc_edge(px, py, pz, src, dst)
    disp_edge = jnp.stack([dx, dy, dz], axis=1)
    dist_edge = _dist_edge_t(dx, dy, dz).T
    zeros = jnp.zeros((N_NODES, 3, N_FEATURES), dtype=jnp.float32)
    return (atom_node, zeros, zeros, disp_edge, dist_edge)


# speedup vs baseline: 3.9738x; 1.0404x over previous
import functools
import math

import jax
import jax.numpy as jnp
from jax import lax
from jax.experimental import pallas as pl
from jax.experimental.pallas import tpu as pltpu
from jax.experimental.pallas import tpu_sc as plsc

N_NODES = 50000
N_EDGES = 800000
N_FEATURES = 128
Z_ROWS = 101
N_BASIS = 16
R_CUT = 5.0

NW = 32

ROWS_PER_W = 1568
ROW_CHUNK = 224
N_ROW_CHUNKS = ROWS_PER_W // ROW_CHUNK

EDGES_PER_W = 25088
EDGE_CHUNK = 1792
N_EDGE_CHUNKS = EDGES_PER_W // EDGE_CHUNK
VECS_PER_CHUNK = EDGE_CHUNK // 16


def _sc_atom_body(z_hbm, emb_hbm, atom_hbm, z_v, table_v,
                  rows_v0, rows_v1, osem0, osem1):
    cid = lax.axis_index("c")
    sid = lax.axis_index("s")
    wid = sid * 2 + cid
    row0 = jnp.minimum(wid * ROWS_PER_W, N_NODES - ROWS_PER_W)
    pltpu.sync_copy(emb_hbm, table_v)
    pltpu.sync_copy(z_hbm.at[pl.ds(row0, ROWS_PER_W)], z_v)
    lane1 = lax.iota(jnp.int32, 16)

    rows = (rows_v0, rows_v1)
    osem = (osem0, osem1)
    ocp = [None, None]
    for k in range(N_ROW_CHUNKS):
        b = k % 2
        if ocp[b] is not None:
            ocp[b].wait()
        rv = rows[b]

        @plsc.parallel_loop(0, (ROW_CHUNK // 16) * (N_FEATURES // 16),
                            step=1, unroll=1)
        def copy_rows(i, _k=k, _rv=rv):
            g = i // (N_FEATURES // 16)
            cblk = (i % (N_FEATURES // 16)) * 16
            zvec = z_v[pl.ds(_k * ROW_CHUNK + g * 16, 16)]
            rowlanes = lane1 + g * 16
            csp = jnp.zeros((16,), jnp.int32) + cblk
            for cj in range(16):
                vals = plsc.load_gather(table_v, [zvec, csp + cj])
                plsc.store_scatter(_rv, [rowlanes, csp + cj], vals)

        ocp[b] = pltpu.async_copy(
            rv, atom_hbm.at[pl.ds(row0 + k * ROW_CHUNK, ROW_CHUNK)], osem[b])
    ocp[0].wait()
    ocp[1].wait()


_sc_atom = functools.partial(
    pl.kernel,
    out_type=jax.ShapeDtypeStruct((N_NODES, N_FEATURES), jnp.float32),
    mesh=plsc.VectorSubcoreMesh(core_axis_name="c", subcore_axis_name="s"),
    compiler_params=pltpu.CompilerParams(needs_layout_passes=False),
    scratch_types=[
        pltpu.VMEM((ROWS_PER_W,), jnp.int32),
        pltpu.VMEM((Z_ROWS, N_FEATURES), jnp.float32),
        pltpu.VMEM((ROW_CHUNK, N_FEATURES), jnp.float32),
        pltpu.VMEM((ROW_CHUNK, N_FEATURES), jnp.float32),
        pltpu.SemaphoreType.DMA,
        pltpu.SemaphoreType.DMA,
    ],
)(_sc_atom_body)


def _prep_pos_body(pos_ref, px_ref, py_ref, pz_ref):
    p = pos_ref[...]
    px_ref[...] = p[:, 0]
    py_ref[...] = p[:, 1]
    pz_ref[...] = p[:, 2]


def _prep_edge_body(e_ref, src_ref, dst_ref):
    e = e_ref[...]
    src_ref[...] = e[0, :]
    dst_ref[...] = e[1, :]


_POS_BLK = 5120


def _prep_pos(pos):
    nblk = -(-N_NODES // _POS_BLK)
    o = jax.ShapeDtypeStruct((N_NODES,), jnp.float32)
    ospec = pl.BlockSpec((_POS_BLK,), lambda i: (i,))
    return pl.pallas_call(
        _prep_pos_body,
        grid=(nblk,),
        in_specs=[pl.BlockSpec((_POS_BLK, 3), lambda i: (i, 0))],
        out_specs=[ospec, ospec, ospec],
        out_shape=[o, o, o],
    )(pos)


_EIDX_BLK = 102400


def _prep_edge(edge_index):
    nblk = -(-N_EDGES // _EIDX_BLK)
    o = jax.ShapeDtypeStruct((N_EDGES,), jnp.int32)
    ospec = pl.BlockSpec((_EIDX_BLK,), lambda i: (i,))
    return pl.pallas_call(
        _prep_edge_body,
        grid=(nblk,),
        in_specs=[pl.BlockSpec((2, _EIDX_BLK), lambda i: (0, i))],
        out_specs=[ospec, ospec],
        out_shape=[o, o],
    )(edge_index)


def _sc_edge_body(px_hbm, py_hbm, pz_hbm, src_hbm, dst_hbm,
                  dx_hbm, dy_hbm, dz_hbm,
                  pcomp_v, src_v, dst_v, out_v):
    cid = lax.axis_index("c")
    sid = lax.axis_index("s")
    wid = sid * 2 + cid
    eb = jnp.minimum(wid * EDGES_PER_W, N_EDGES - EDGES_PER_W)
    pltpu.sync_copy(src_hbm.at[pl.ds(eb, EDGES_PER_W)], src_v)
    pltpu.sync_copy(dst_hbm.at[pl.ds(eb, EDGES_PER_W)], dst_v)
    for p_hbm, o_hbm in ((px_hbm, dx_hbm), (py_hbm, dy_hbm), (pz_hbm, dz_hbm)):
        pltpu.sync_copy(p_hbm, pcomp_v)

        @plsc.parallel_loop(0, EDGES_PER_W, step=16, unroll=8)
        def body(i):
            s = src_v[pl.ds(i, 16)]
            t = dst_v[pl.ds(i, 16)]
            out_v[pl.ds(i, 16)] = (plsc.load_gather(pcomp_v, [s])
                                   - plsc.load_gather(pcomp_v, [t]))

        pltpu.sync_copy(out_v, o_hbm.at[pl.ds(eb, EDGES_PER_W)])


_sc_edge = functools.partial(
    pl.kernel,
    out_type=(
        jax.ShapeDtypeStruct((N_EDGES,), jnp.float32),
        jax.ShapeDtypeStruct((N_EDGES,), jnp.float32),
        jax.ShapeDtypeStruct((N_EDGES,), jnp.float32),
    ),
    mesh=plsc.VectorSubcoreMesh(core_axis_name="c", subcore_axis_name="s"),
    compiler_params=pltpu.CompilerParams(needs_layout_passes=False),
    scratch_types=[
        pltpu.VMEM((N_NODES,), jnp.float32),
        pltpu.VMEM((EDGES_PER_W,), jnp.int32),
        pltpu.VMEM((EDGES_PER_W,), jnp.int32),
        pltpu.VMEM((EDGES_PER_W,), jnp.float32),
    ],
)(_sc_edge_body)


_EDGE_BLK = 5120
_N_EDGE_BLKS = -(-N_EDGES // _EDGE_BLK)
_CENTER_STEP = R_CUT / (N_BASIS - 1)
_GAMMA = 1.0 / (_CENTER_STEP * _CENTER_STEP)


_COS_POLY = (0.9999999890590233, -4.934801124863485, 4.058694841243486,
             -1.3351584301699686, 0.23502980840174797,
             -0.025358983640522026, 0.001593910683660976)


def _tc_dist_body(dx_ref, dy_ref, dz_ref, out_ref):
    x = dx_ref[...]
    y = dy_ref[...]
    z = dz_ref[...]
    d2 = x * x + y * y + z * z
    dist = jnp.sqrt(d2)
    v = d2 * (1.0 / (R_CUT * R_CUT))
    p = jnp.float32(_COS_POLY[6])
    for coef in _COS_POLY[5::-1]:
        p = p * v + jnp.float32(coef)
    cut = 0.5 * (p + 1.0) * (v < 1.0).astype(jnp.float32)
    cut2 = jnp.broadcast_to(cut.reshape(1, _EDGE_BLK), (N_BASIS, _EDGE_BLK))
    dist2 = jnp.broadcast_to(dist.reshape(1, _EDGE_BLK), (N_BASIS, _EDGE_BLK))
    centers = lax.broadcasted_iota(jnp.int32, (N_BASIS, _EDGE_BLK), 0).astype(
        jnp.float32) * _CENTER_STEP
    delta = dist2 - centers
    out_ref[...] = cut2 * jnp.exp(-_GAMMA * delta * delta)


def _dist_edge_t(dx, dy, dz):
    spec = pl.BlockSpec((_EDGE_BLK,), lambda i: (i,))
    return pl.pallas_call(
        _tc_dist_body,
        grid=(_N_EDGE_BLKS,),
        in_specs=[spec, spec, spec],
        out_specs=pl.BlockSpec((N_BASIS, _EDGE_BLK), lambda i: (0, i)),
        out_shape=jax.ShapeDtypeStruct((N_BASIS, N_EDGES), jnp.float32),
    )(dx, dy, dz)


def kernel(z, pos, edge_index, emb_table):
    px, py, pz = _prep_pos(pos)
    src, dst = _prep_edge(edge_index)
    atom_node = _sc_atom(z, emb_table)
    dx, dy, dz = _sc_edge(px, py, pz, src, dst)
    disp_edge = jnp.stack([dx, dy, dz], axis=1)
    dist_edge = _dist_edge_t(dx, dy, dz).T
    zeros = jnp.zeros((N_NODES, 3, N_FEATURES), dtype=jnp.float32)
    return (atom_node, zeros, zeros, disp_edge, dist_edge)


# speedup vs baseline: 4.0127x; 1.0098x over previous
import functools
import math

import jax
import jax.numpy as jnp
from jax import lax
from jax.experimental import pallas as pl
from jax.experimental.pallas import tpu as pltpu
from jax.experimental.pallas import tpu_sc as plsc

N_NODES = 50000
N_EDGES = 800000
N_FEATURES = 128
Z_ROWS = 101
N_BASIS = 16
R_CUT = 5.0

NW = 32

ROWS_PER_W = 1568
ROW_CHUNK = 224
N_ROW_CHUNKS = ROWS_PER_W // ROW_CHUNK

EDGES_PER_W = 25088
EDGE_CHUNK = 1792
N_EDGE_CHUNKS = EDGES_PER_W // EDGE_CHUNK
VECS_PER_CHUNK = EDGE_CHUNK // 16


def _sc_atom_body(z_hbm, emb_hbm, atom_hbm, z_v, table_v,
                  rows_v0, rows_v1, osem0, osem1):
    cid = lax.axis_index("c")
    sid = lax.axis_index("s")
    wid = sid * 2 + cid
    row0 = jnp.minimum(wid * ROWS_PER_W, N_NODES - ROWS_PER_W)
    pltpu.sync_copy(emb_hbm, table_v)
    pltpu.sync_copy(z_hbm.at[pl.ds(row0, ROWS_PER_W)], z_v)
    lane1 = lax.iota(jnp.int32, 16)

    rows = (rows_v0, rows_v1)
    osem = (osem0, osem1)
    ocp = [None, None]
    for k in range(N_ROW_CHUNKS):
        b = k % 2
        if ocp[b] is not None:
            ocp[b].wait()
        rv = rows[b]

        @plsc.parallel_loop(0, (ROW_CHUNK // 16) * (N_FEATURES // 16),
                            step=1, unroll=4)
        def copy_rows(i, _k=k, _rv=rv):
            g = i // (N_FEATURES // 16)
            cblk = (i % (N_FEATURES // 16)) * 16
            zvec = z_v[pl.ds(_k * ROW_CHUNK + g * 16, 16)]
            rowlanes = lane1 + g * 16
            csp = jnp.zeros((16,), jnp.int32) + cblk
            for cj in range(16):
                col = csp + ((lane1 + cj) & 15)
                vals = plsc.load_gather(table_v, [zvec, col])
                plsc.store_scatter(_rv, [rowlanes, col], vals)

        ocp[b] = pltpu.async_copy(
            rv, atom_hbm.at[pl.ds(row0 + k * ROW_CHUNK, ROW_CHUNK)], osem[b])
    ocp[0].wait()
    ocp[1].wait()


_sc_atom = functools.partial(
    pl.kernel,
    out_type=jax.ShapeDtypeStruct((N_NODES, N_FEATURES), jnp.float32),
    mesh=plsc.VectorSubcoreMesh(core_axis_name="c", subcore_axis_name="s"),
    compiler_params=pltpu.CompilerParams(needs_layout_passes=False),
    scratch_types=[
        pltpu.VMEM((ROWS_PER_W,), jnp.int32),
        pltpu.VMEM((Z_ROWS, N_FEATURES), jnp.float32),
        pltpu.VMEM((ROW_CHUNK, N_FEATURES), jnp.float32),
        pltpu.VMEM((ROW_CHUNK, N_FEATURES), jnp.float32),
        pltpu.SemaphoreType.DMA,
        pltpu.SemaphoreType.DMA,
    ],
)(_sc_atom_body)


def _prep_pos_body(pos_ref, px_ref, py_ref, pz_ref):
    p = pos_ref[...]
    px_ref[...] = p[:, 0]
    py_ref[...] = p[:, 1]
    pz_ref[...] = p[:, 2]


def _prep_edge_body(e_ref, src_ref, dst_ref):
    e = e_ref[...]
    src_ref[...] = e[0, :]
    dst_ref[...] = e[1, :]


_POS_BLK = 5120


def _prep_pos(pos):
    nblk = -(-N_NODES // _POS_BLK)
    o = jax.ShapeDtypeStruct((N_NODES,), jnp.float32)
    ospec = pl.BlockSpec((_POS_BLK,), lambda i: (i,))
    return pl.pallas_call(
        _prep_pos_body,
        grid=(nblk,),
        in_specs=[pl.BlockSpec((_POS_BLK, 3), lambda i: (i, 0))],
        out_specs=[ospec, ospec, ospec],
        out_shape=[o, o, o],
    )(pos)


_EIDX_BLK = 102400


def _prep_edge(edge_index):
    nblk = -(-N_EDGES // _EIDX_BLK)
    o = jax.ShapeDtypeStruct((N_EDGES,), jnp.int32)
    ospec = pl.BlockSpec((_EIDX_BLK,), lambda i: (i,))
    return pl.pallas_call(
        _prep_edge_body,
        grid=(nblk,),
        in_specs=[pl.BlockSpec((2, _EIDX_BLK), lambda i: (0, i))],
        out_specs=[ospec, ospec],
        out_shape=[o, o],
    )(edge_index)


def _sc_edge_body(px_hbm, py_hbm, pz_hbm, src_hbm, dst_hbm,
                  dx_hbm, dy_hbm, dz_hbm,
                  pcomp_v, src_v, dst_v, out_v):
    cid = lax.axis_index("c")
    sid = lax.axis_index("s")
    wid = sid * 2 + cid
    eb = jnp.minimum(wid * EDGES_PER_W, N_EDGES - EDGES_PER_W)
    pltpu.sync_copy(src_hbm.at[pl.ds(eb, EDGES_PER_W)], src_v)
    pltpu.sync_copy(dst_hbm.at[pl.ds(eb, EDGES_PER_W)], dst_v)
    for p_hbm, o_hbm in ((px_hbm, dx_hbm), (py_hbm, dy_hbm), (pz_hbm, dz_hbm)):
        pltpu.sync_copy(p_hbm, pcomp_v)

        @plsc.parallel_loop(0, EDGES_PER_W, step=16, unroll=8)
        def body(i):
            s = src_v[pl.ds(i, 16)]
            t = dst_v[pl.ds(i, 16)]
            out_v[pl.ds(i, 16)] = (plsc.load_gather(pcomp_v, [s])
                                   - plsc.load_gather(pcomp_v, [t]))

        pltpu.sync_copy(out_v, o_hbm.at[pl.ds(eb, EDGES_PER_W)])


_sc_edge = functools.partial(
    pl.kernel,
    out_type=(
        jax.ShapeDtypeStruct((N_EDGES,), jnp.float32),
        jax.ShapeDtypeStruct((N_EDGES,), jnp.float32),
        jax.ShapeDtypeStruct((N_EDGES,), jnp.float32),
    ),
    mesh=plsc.VectorSubcoreMesh(core_axis_name="c", subcore_axis_name="s"),
    compiler_params=pltpu.CompilerParams(needs_layout_passes=False),
    scratch_types=[
        pltpu.VMEM((N_NODES,), jnp.float32),
        pltpu.VMEM((EDGES_PER_W,), jnp.int32),
        pltpu.VMEM((EDGES_PER_W,), jnp.int32),
        pltpu.VMEM((EDGES_PER_W,), jnp.float32),
    ],
)(_sc_edge_body)


_EDGE_BLK = 5120
_N_EDGE_BLKS = -(-N_EDGES // _EDGE_BLK)
_CENTER_STEP = R_CUT / (N_BASIS - 1)
_GAMMA = 1.0 / (_CENTER_STEP * _CENTER_STEP)


_COS_POLY = (0.9999999890590233, -4.934801124863485, 4.058694841243486,
             -1.3351584301699686, 0.23502980840174797,
             -0.025358983640522026, 0.001593910683660976)


def _tc_dist_body(dx_ref, dy_ref, dz_ref, out_ref):
    x = dx_ref[...]
    y = dy_ref[...]
    z = dz_ref[...]
    d2 = x * x + y * y + z * z
    dist = jnp.sqrt(d2)
    v = d2 * (1.0 / (R_CUT * R_CUT))
    p = jnp.float32(_COS_POLY[6])
    for coef in _COS_POLY[5::-1]:
        p = p * v + jnp.float32(coef)
    cut = 0.5 * (p + 1.0) * (v < 1.0).astype(jnp.float32)
    cut2 = jnp.broadcast_to(cut.reshape(1, _EDGE_BLK), (N_BASIS, _EDGE_BLK))
    dist2 = jnp.broadcast_to(dist.reshape(1, _EDGE_BLK), (N_BASIS, _EDGE_BLK))
    centers = lax.broadcasted_iota(jnp.int32, (N_BASIS, _EDGE_BLK), 0).astype(
        jnp.float32) * _CENTER_STEP
    delta = dist2 - centers
    out_ref[...] = cut2 * jnp.exp(-_GAMMA * delta * delta)


def _dist_edge_t(dx, dy, dz):
    spec = pl.BlockSpec((_EDGE_BLK,), lambda i: (i,))
    return pl.pallas_call(
        _tc_dist_body,
        grid=(_N_EDGE_BLKS,),
        in_specs=[spec, spec, spec],
        out_specs=pl.BlockSpec((N_BASIS, _EDGE_BLK), lambda i: (0, i)),
        out_shape=jax.ShapeDtypeStruct((N_BASIS, N_EDGES), jnp.float32),
    )(dx, dy, dz)


def kernel(z, pos, edge_index, emb_table):
    px, py, pz = _prep_pos(pos)
    src, dst = _prep_edge(edge_index)
    atom_node = _sc_atom(z, emb_table)
    dx, dy, dz = _sc_edge(px, py, pz, src, dst)
    disp_edge = jnp.stack([dx, dy, dz], axis=1)
    dist_edge = _dist_edge_t(dx, dy, dz).T
    zeros = jnp.zeros((N_NODES, 3, N_FEATURES), dtype=jnp.float32)
    return (atom_node, zeros, zeros, disp_edge, dist_edge)
